# Initial kernel scaffold; baseline (speedup 1.0000x reference)
import jax
import jax.numpy as jnp
from jax.experimental import pallas as pl


def kernel(x, edge_index, W1, b1, W2, b2, Wp, bp, Wc, bc):
    raise NotImplementedError("write your pallas kernel here")


# speedup vs baseline: 11.9399x; 11.9399x over previous
import functools

import jax
import jax.numpy as jnp
from jax import lax
from jax.experimental import pallas as pl
from jax.experimental.pallas import tpu as pltpu
from jax.experimental.pallas import tpu_sc as plsc

N = 10000
DIN = 128
H = 32
C = 10
NCLS = 7
E = 160000

NC = 2
NS = 16
NW = NC * NS
L = 16

NPAD = 10112
RPW = NPAD // NS
CSZ = 128
EPT = 5120
CH = EPT // CSZ
EP = NW * EPT

BL = 1000
GRID = N // BL

_MESH = dict(core_axis_name="c", subcore_axis_name="s", num_cores=NC,
             num_subcores=NS)


def _make_sc_agg(W):
    mesh = plsc.VectorSubcoreMesh(**_MESH)

    @functools.partial(
        pl.kernel,
        out_type=jax.ShapeDtypeStruct((NC * NPAD, W), jnp.float32),
        mesh=mesh,
        compiler_params=pltpu.CompilerParams(use_tc_tiling_on_sc=False),
        scratch_types=[
            pltpu.VMEM((CH, CSZ), jnp.int32),
            pltpu.VMEM((CH, CSZ), jnp.int32),
            pltpu.VMEM((CSZ, W), jnp.float32),
            pltpu.VMEM((RPW, W), jnp.float32),
            pltpu.VMEM_SHARED((NPAD, W), jnp.float32),
            pltpu.SemaphoreType.DMA,
        ],
    )
    def k(table_hbm, gidx_hbm, sidx_hbm, out_hbm, gi_v, si_v, rows_v,
          stage_v, acc_sh, sem):
        cid = lax.axis_index("c")
        sid = lax.axis_index("s")
        wid = sid * NC + cid
        pltpu.sync_copy(gidx_hbm.at[wid], gi_v)
        pltpu.sync_copy(sidx_hbm.at[wid], si_v)
        zero = jnp.zeros((L,), jnp.float32)

        def fz(i, _):
            for w0 in range(W // L):
                stage_v[i, pl.ds(w0 * L, L)] = zero
            return 0

        lax.fori_loop(0, RPW, fz, 0)
        pltpu.sync_copy(stage_v, acc_sh.at[pl.ds(sid * RPW, RPW)])
        plsc.subcore_barrier()

        def chunk(j, _):
            pltpu.async_copy(table_hbm.at[gi_v.at[j]], rows_v, sem).wait()
            pltpu.sync_copy(rows_v, acc_sh.at[si_v.at[j]], add=True)
            return 0

        lax.fori_loop(0, CH, chunk, 0)
        plsc.subcore_barrier()
        pltpu.sync_copy(
            acc_sh.at[pl.ds(sid * RPW, RPW)],
            out_hbm.at[pl.ds(cid * NPAD + sid * RPW, RPW)],
        )

    return k


def _make_sc_edge():
    mesh = plsc.VectorSubcoreMesh(**_MESH)

    @functools.partial(
        pl.kernel,
        out_type=jax.ShapeDtypeStruct((NW * 2, L), jnp.float32),
        mesh=mesh,
        compiler_params=pltpu.CompilerParams(use_tc_tiling_on_sc=False),
        scratch_types=[
            pltpu.VMEM((CH, CSZ), jnp.int32),
            pltpu.VMEM((CH, CSZ), jnp.int32),
            pltpu.VMEM((CSZ, L), jnp.float32),
            pltpu.VMEM((CSZ, L), jnp.float32),
            pltpu.VMEM((2, L), jnp.float32),
            pltpu.SemaphoreType.DMA,
            pltpu.SemaphoreType.DMA,
        ],
    )
    def k(st_hbm, sidx_hbm, didx_hbm, out_hbm, si_v, di_v, ra_v, rb_v,
          res_v, sema, semb):
        cid = lax.axis_index("c")
        sid = lax.axis_index("s")
        wid = sid * NC + cid
        pltpu.sync_copy(sidx_hbm.at[wid], si_v)
        pltpu.sync_copy(didx_hbm.at[wid], di_v)
        zero = jnp.zeros((L,), jnp.float32)

        def chunk(j, carry):
            num, den = carry
            cpa = pltpu.async_copy(st_hbm.at[si_v.at[j]], ra_v, sema)
            cpb = pltpu.async_copy(st_hbm.at[di_v.at[j]], rb_v, semb)
            cpa.wait()
            cpb.wait()

            def inner(r, c2):
                n2, d2 = c2
                va = ra_v[r]
                vb = rb_v[r]
                return (n2 + va * vb, d2 + va * va)

            return lax.fori_loop(0, CSZ, inner, (num, den))

        num, den = lax.fori_loop(0, CH, chunk, (zero, zero))
        res_v[0] = num
        res_v[1] = den
        pltpu.sync_copy(res_v, out_hbm.at[pl.ds(wid * 2, 2)])

    return k


_sc_deg = _make_sc_agg(L)
_sc_agg = _make_sc_agg(H)
_sc_edge = _make_sc_edge()


def _tc_b_body(x_ref, w1_ref, dp0_ref, dp1_ref, u1_ref, dis_ref):
    deg = dp0_ref[...] + dp1_ref[...] + 1.0
    dis = lax.rsqrt(deg)
    h1 = jnp.dot(x_ref[...], w1_ref[...], preferred_element_type=jnp.float32)
    u1_ref[...] = dis[:, 0:1] * h1
    dis_ref[...] = dis


def _tc_b(x, w1, dp0, dp1):
    return pl.pallas_call(
        _tc_b_body,
        grid=(GRID,),
        in_specs=[
            pl.BlockSpec((BL, DIN), lambda i: (i, 0)),
            pl.BlockSpec((DIN, H), lambda i: (0, 0)),
            pl.BlockSpec((BL, L), lambda i: (i, 0)),
            pl.BlockSpec((BL, L), lambda i: (i, 0)),
        ],
        out_specs=[
            pl.BlockSpec((BL, H), lambda i: (i, 0)),
            pl.BlockSpec((BL, L), lambda i: (i, 0)),
        ],
        out_shape=[
            jax.ShapeDtypeStruct((N, H), jnp.float32),
            jax.ShapeDtypeStruct((N, L), jnp.float32),
        ],
    )(x, w1, dp0, dp1)


def _tc_c_body(p0_ref, p1_ref, u1_ref, dis_ref, b1_ref, w2_ref, u2_ref):
    d0 = dis_ref[...][:, 0:1]
    h = jnp.maximum(
        d0 * (p0_ref[...] + p1_ref[...] + u1_ref[...]) + b1_ref[...], 0.0)
    u2_ref[...] = d0 * jnp.dot(h, w2_ref[...],
                               preferred_element_type=jnp.float32)


def _tc_c(p0, p1, u1, dis, b1r, w2):
    return pl.pallas_call(
        _tc_c_body,
        grid=(GRID,),
        in_specs=[
            pl.BlockSpec((BL, H), lambda i: (i, 0)),
            pl.BlockSpec((BL, H), lambda i: (i, 0)),
            pl.BlockSpec((BL, H), lambda i: (i, 0)),
            pl.BlockSpec((BL, L), lambda i: (i, 0)),
            pl.BlockSpec((1, H), lambda i: (0, 0)),
            pl.BlockSpec((H, H), lambda i: (0, 0)),
        ],
        out_specs=pl.BlockSpec((BL, H), lambda i: (i, 0)),
        out_shape=jax.ShapeDtypeStruct((N, H), jnp.float32),
    )(p0, p1, u1, dis, b1r, w2)


def _tc_d_body(q0_ref, q1_ref, u2_ref, dis_ref, b2_ref, wp_ref, bp_ref,
               wc_ref, bc_ref, st_ref, ss_ref, lp_ref):
    i = pl.program_id(0)
    d0 = dis_ref[...][:, 0:1]
    h = jnp.maximum(
        d0 * (q0_ref[...] + q1_ref[...] + u2_ref[...]) + b2_ref[...], 0.0)
    sl = jnp.dot(h, wp_ref[...], preferred_element_type=jnp.float32) \
        + bp_ref[...]
    m = jnp.max(sl, axis=1, keepdims=True)
    p = jnp.exp(sl - m)
    s = p / jnp.sum(p, axis=1, keepdims=True)
    st_ref[...] = jnp.concatenate(
        [s, jnp.ones((BL, 1), jnp.float32), jnp.zeros((BL, 5), jnp.float32)],
        axis=1)
    ssb = lax.dot_general(s, s, (((0,), (0,)), ((), ())),
                          preferred_element_type=jnp.float32)

    @pl.when(i == 0)
    def _():
        ss_ref[...] = jnp.zeros_like(ss_ref)

    ss_ref[...] += ssb
    lo = jnp.dot(h, wc_ref[...], preferred_element_type=jnp.float32) \
        + bc_ref[...]
    mm = jnp.max(lo, axis=1, keepdims=True)
    lp_ref[...] = lo - mm - jnp.log(
        jnp.sum(jnp.exp(lo - mm), axis=1, keepdims=True))


def _tc_d(q0, q1, u2, dis, b2r, wp, bpr, wc, bcr):
    return pl.pallas_call(
        _tc_d_body,
        grid=(GRID,),
        in_specs=[
            pl.BlockSpec((BL, H), lambda i: (i, 0)),
            pl.BlockSpec((BL, H), lambda i: (i, 0)),
            pl.BlockSpec((BL, H), lambda i: (i, 0)),
            pl.BlockSpec((BL, L), lambda i: (i, 0)),
            pl.BlockSpec((1, H), lambda i: (0, 0)),
            pl.BlockSpec((H, C), lambda i: (0, 0)),
            pl.BlockSpec((1, C), lambda i: (0, 0)),
            pl.BlockSpec((H, NCLS), lambda i: (0, 0)),
            pl.BlockSpec((1, NCLS), lambda i: (0, 0)),
        ],
        out_specs=[
            pl.BlockSpec((BL, L), lambda i: (i, 0)),
            pl.BlockSpec((C, C), lambda i: (0, 0)),
            pl.BlockSpec((BL, NCLS), lambda i: (i, 0)),
        ],
        out_shape=[
            jax.ShapeDtypeStruct((N, L), jnp.float32),
            jax.ShapeDtypeStruct((C, C), jnp.float32),
            jax.ShapeDtypeStruct((N, NCLS), jnp.float32),
        ],
    )(q0, q1, u2, dis, b2r, wp, bpr, wc, bcr)


def _tc_f_body(na_ref, da_ref, ss_ref, mc_ref, o_ref):
    num = jnp.sum(na_ref[...]) - float(E)
    den = jnp.sum(da_ref[...]) - float(E)
    mc_ref[...] = jnp.full((1, 1), -(num / den), jnp.float32)
    ssv = ss_ref[...]
    nss = jnp.sqrt(jnp.sum(ssv * ssv))
    eye = (lax.broadcasted_iota(jnp.int32, (C, C), 0)
           == lax.broadcasted_iota(jnp.int32, (C, C), 1)).astype(jnp.float32)
    t = ssv / nss - eye / jnp.sqrt(jnp.float32(C))
    o_ref[...] = jnp.full((1, 1), jnp.sqrt(jnp.sum(t * t)), jnp.float32)


def _tc_f(na, da, ss):
    return pl.pallas_call(
        _tc_f_body,
        out_shape=[
            jax.ShapeDtypeStruct((1, 1), jnp.float32),
            jax.ShapeDtypeStruct((1, 1), jnp.float32),
        ],
    )(na, da, ss)


def kernel(x, edge_index, W1, b1, W2, b2, Wp, bp, Wc, bc):
    src = edge_index[0]
    dst = edge_index[1]
    pad = jnp.full((EP - E,), N, jnp.int32)
    srcp = jnp.concatenate([src, pad]).reshape(NW, CH, CSZ)
    dstp = jnp.concatenate([dst, pad]).reshape(NW, CH, CSZ)

    ones_tab = jnp.ones((NPAD, L), jnp.float32)
    degp = _sc_deg(ones_tab, dstp, dstp).reshape(NC, NPAD, L)

    u1, dis = _tc_b(x, W1, degp[0, :N], degp[1, :N])

    u1p = jnp.pad(u1, ((0, NPAD - N), (0, 0)))
    a1 = _sc_agg(u1p, srcp, dstp).reshape(NC, NPAD, H)

    u2 = _tc_c(a1[0, :N], a1[1, :N], u1, dis, b1.reshape(1, H), W2)

    u2p = jnp.pad(u2, ((0, NPAD - N), (0, 0)))
    a2 = _sc_agg(u2p, srcp, dstp).reshape(NC, NPAD, H)

    st, ss, logp = _tc_d(a2[0, :N], a2[1, :N], u2, dis, b2.reshape(1, H),
                         Wp, bp.reshape(1, C), Wc, bc.reshape(1, NCLS))

    stp = jnp.pad(st, ((0, NPAD - N), (0, 0)))
    eacc = _sc_edge(stp, srcp, dstp).reshape(NW, 2, L)

    mc, o = _tc_f(eacc[:, 0], eacc[:, 1], ss)
    return (logp, jnp.reshape(mc, ()), jnp.reshape(o, ()))


# speedup vs baseline: 24.7430x; 2.0723x over previous
import functools

import jax
import jax.numpy as jnp
from jax import lax
from jax.experimental import pallas as pl
from jax.experimental.pallas import tpu as pltpu
from jax.experimental.pallas import tpu_sc as plsc

N = 10000
DIN = 128
H = 32
C = 10
NCLS = 7
E = 160000

NC = 2
NS = 16
NW = NC * NS
L = 16

NPAD = 10112
RPW = NPAD // NS
CSZ = 128
EPT = 5120
CH = EPT // CSZ
CH_LAST = (E - (NW - 1) * EPT) // CSZ
EP = NW * EPT

BL = 1000
GRID = N // BL

_MESH = dict(core_axis_name="c", subcore_axis_name="s", num_cores=NC,
             num_subcores=NS)
_SC_PARAMS = pltpu.CompilerParams(use_tc_tiling_on_sc=False)


def _nchunks(wid):
    return jnp.where(wid == NW - 1, CH_LAST, CH)


def _make_sc_deg():
    mesh = plsc.VectorSubcoreMesh(**_MESH)

    @functools.partial(
        pl.kernel,
        out_type=jax.ShapeDtypeStruct((NC * NPAD, L), jnp.float32),
        mesh=mesh,
        compiler_params=_SC_PARAMS,
        scratch_types=[
            pltpu.VMEM((CH, CSZ), jnp.int32),
            pltpu.VMEM((CSZ, L), jnp.float32),
            pltpu.VMEM((RPW, L), jnp.float32),
            pltpu.VMEM_SHARED((NPAD, L), jnp.float32),
            pltpu.SemaphoreType.DMA,
        ],
    )
    def k(sidx_hbm, out_hbm, si_v, ones_v, stage_v, acc_sh, sem):
        cid = lax.axis_index("c")
        sid = lax.axis_index("s")
        wid = sid * NC + cid
        pltpu.sync_copy(sidx_hbm.at[wid], si_v)
        one = jnp.full((L,), 1.0, jnp.float32)
        zero = jnp.zeros((L,), jnp.float32)

        def f1(i, _):
            ones_v[i, :] = one
            return 0

        lax.fori_loop(0, CSZ, f1, 0)

        def fz(i, _):
            stage_v[i, :] = zero
            return 0

        lax.fori_loop(0, RPW, fz, 0)
        pltpu.sync_copy(stage_v, acc_sh.at[pl.ds(sid * RPW, RPW)])
        plsc.subcore_barrier()
        nchk = _nchunks(wid)

        def fire(j, _):
            pltpu.async_copy(ones_v, acc_sh.at[si_v.at[j]], sem, add=True)
            return 0

        lax.fori_loop(0, nchk, fire, 0)

        def drain(j, _):
            pltpu.make_async_copy(ones_v, acc_sh.at[si_v.at[j]], sem).wait()
            return 0

        lax.fori_loop(0, nchk, drain, 0)
        plsc.subcore_barrier()
        pltpu.sync_copy(
            acc_sh.at[pl.ds(sid * RPW, RPW)],
            out_hbm.at[pl.ds(cid * NPAD + sid * RPW, RPW)],
        )

    return k


def _make_sc_agg(W):
    mesh = plsc.VectorSubcoreMesh(**_MESH)

    @functools.partial(
        pl.kernel,
        out_type=jax.ShapeDtypeStruct((NC * NPAD, W), jnp.float32),
        mesh=mesh,
        compiler_params=_SC_PARAMS,
        scratch_types=[
            pltpu.VMEM((CH, CSZ), jnp.int32),
            pltpu.VMEM((CH, CSZ), jnp.int32),
            pltpu.VMEM((CSZ, W), jnp.float32),
            pltpu.VMEM((CSZ, W), jnp.float32),
            pltpu.VMEM((RPW, W), jnp.float32),
            pltpu.VMEM_SHARED((NPAD, W), jnp.float32),
            pltpu.SemaphoreType.DMA,
            pltpu.SemaphoreType.DMA,
        ],
    )
    def k(table_hbm, gidx_hbm, sidx_hbm, out_hbm, gi_v, si_v, rows0, rows1,
          stage_v, acc_sh, semg0, semg1):
        cid = lax.axis_index("c")
        sid = lax.axis_index("s")
        wid = sid * NC + cid
        pltpu.sync_copy(gidx_hbm.at[wid], gi_v)
        pltpu.sync_copy(sidx_hbm.at[wid], si_v)
        zero = jnp.zeros((L,), jnp.float32)

        def fz(i, _):
            for w0 in range(W // L):
                stage_v[i, pl.ds(w0 * L, L)] = zero
            return 0

        lax.fori_loop(0, RPW, fz, 0)
        pltpu.sync_copy(stage_v, acc_sh.at[pl.ds(sid * RPW, RPW)])
        plsc.subcore_barrier()
        nchk = _nchunks(wid)
        nt = nchk // 2

        def body(t, _):
            j0 = 2 * t
            j1 = j0 + 1
            cpa = pltpu.async_copy(table_hbm.at[gi_v.at[j0]], rows0, semg0)
            cpb = pltpu.async_copy(table_hbm.at[gi_v.at[j1]], rows1, semg1)
            cpa.wait()
            pltpu.sync_copy(rows0, acc_sh.at[si_v.at[j0]], add=True)
            cpb.wait()
            pltpu.sync_copy(rows1, acc_sh.at[si_v.at[j1]], add=True)
            return 0

        lax.fori_loop(0, nt, body, 0)
        plsc.subcore_barrier()
        pltpu.sync_copy(
            acc_sh.at[pl.ds(sid * RPW, RPW)],
            out_hbm.at[pl.ds(cid * NPAD + sid * RPW, RPW)],
        )

    return k


def _make_sc_edge():
    mesh = plsc.VectorSubcoreMesh(**_MESH)

    @functools.partial(
        pl.kernel,
        out_type=jax.ShapeDtypeStruct((NW * 2, L), jnp.float32),
        mesh=mesh,
        compiler_params=_SC_PARAMS,
        scratch_types=[
            pltpu.VMEM((CH, CSZ), jnp.int32),
            pltpu.VMEM((CH, CSZ), jnp.int32),
            pltpu.VMEM((CSZ, L), jnp.float32),
            pltpu.VMEM((CSZ, L), jnp.float32),
            pltpu.VMEM((CSZ, L), jnp.float32),
            pltpu.VMEM((CSZ, L), jnp.float32),
            pltpu.VMEM((2, L), jnp.float32),
            pltpu.SemaphoreType.DMA,
            pltpu.SemaphoreType.DMA,
        ],
    )
    def k(st_hbm, sidx_hbm, didx_hbm, out_hbm, si_v, di_v, ra0, rb0, ra1,
          rb1, res_v, sem0, sem1):
        cid = lax.axis_index("c")
        sid = lax.axis_index("s")
        wid = sid * NC + cid
        pltpu.sync_copy(sidx_hbm.at[wid], si_v)
        pltpu.sync_copy(didx_hbm.at[wid], di_v)
        zero = jnp.zeros((L,), jnp.float32)
        nchk = _nchunks(wid)
        nt = nchk // 2

        def accum(ra, rb, carry):
            def inner(r8, c2):
                accs = list(c2)
                base = r8 * 8
                for u in range(8):
                    va = ra[base + u]
                    vb = rb[base + u]
                    p = u % 4
                    accs[p] = accs[p] + va * vb
                    accs[4 + p] = accs[4 + p] + va * va
                return tuple(accs)

            return lax.fori_loop(0, CSZ // 8, inner, carry)

        def body(t, carry):
            j0 = 2 * t
            j1 = j0 + 1
            cpa0 = pltpu.async_copy(st_hbm.at[si_v.at[j0]], ra0, sem0)
            cpb0 = pltpu.async_copy(st_hbm.at[di_v.at[j0]], rb0, sem0)
            cpa1 = pltpu.async_copy(st_hbm.at[si_v.at[j1]], ra1, sem1)
            cpb1 = pltpu.async_copy(st_hbm.at[di_v.at[j1]], rb1, sem1)
            cpa0.wait()
            cpb0.wait()
            carry = accum(ra0, rb0, carry)
            cpa1.wait()
            cpb1.wait()
            return accum(ra1, rb1, carry)

        init = tuple(zero for _ in range(8))
        accs = lax.fori_loop(0, nt, body, init)
        res_v[0] = accs[0] + accs[1] + accs[2] + accs[3]
        res_v[1] = accs[4] + accs[5] + accs[6] + accs[7]
        pltpu.sync_copy(res_v, out_hbm.at[pl.ds(wid * 2, 2)])

    return k


_sc_deg = _make_sc_deg()
_sc_agg = _make_sc_agg(H)
_sc_edge = _make_sc_edge()


def _tc_b_body(x_ref, w1_ref, dp_ref, u1_ref, dis_ref):
    deg = dp_ref[0] + dp_ref[1] + 1.0
    dis = lax.rsqrt(deg)
    h1 = jnp.dot(x_ref[...], w1_ref[...], preferred_element_type=jnp.float32)
    u1_ref[...] = dis[:, 0:1] * h1
    dis_ref[...] = dis


def _tc_b(x, w1, degp):
    return pl.pallas_call(
        _tc_b_body,
        grid=(GRID,),
        in_specs=[
            pl.BlockSpec((BL, DIN), lambda i: (i, 0)),
            pl.BlockSpec((DIN, H), lambda i: (0, 0)),
            pl.BlockSpec((NC, BL, L), lambda i: (0, i, 0)),
        ],
        out_specs=[
            pl.BlockSpec((BL, H), lambda i: (i, 0)),
            pl.BlockSpec((BL, L), lambda i: (i, 0)),
        ],
        out_shape=[
            jax.ShapeDtypeStruct((N, H), jnp.float32),
            jax.ShapeDtypeStruct((N, L), jnp.float32),
        ],
    )(x, w1, degp)


def _tc_c_body(ap_ref, u1_ref, dis_ref, b1_ref, w2_ref, u2_ref):
    d0 = dis_ref[...][:, 0:1]
    h = jnp.maximum(
        d0 * (ap_ref[0] + ap_ref[1] + u1_ref[...]) + b1_ref[...], 0.0)
    u2_ref[...] = d0 * jnp.dot(h, w2_ref[...],
                               preferred_element_type=jnp.float32)


def _tc_c(ap, u1, dis, b1r, w2):
    return pl.pallas_call(
        _tc_c_body,
        grid=(GRID,),
        in_specs=[
            pl.BlockSpec((NC, BL, H), lambda i: (0, i, 0)),
            pl.BlockSpec((BL, H), lambda i: (i, 0)),
            pl.BlockSpec((BL, L), lambda i: (i, 0)),
            pl.BlockSpec((1, H), lambda i: (0, 0)),
            pl.BlockSpec((H, H), lambda i: (0, 0)),
        ],
        out_specs=pl.BlockSpec((BL, H), lambda i: (i, 0)),
        out_shape=jax.ShapeDtypeStruct((N, H), jnp.float32),
    )(ap, u1, dis, b1r, w2)


def _tc_d_body(ap_ref, u2_ref, dis_ref, b2_ref, wp_ref, bp_ref,
               wc_ref, bc_ref, st_ref, ss_ref, lp_ref):
    i = pl.program_id(0)
    d0 = dis_ref[...][:, 0:1]
    h = jnp.maximum(
        d0 * (ap_ref[0] + ap_ref[1] + u2_ref[...]) + b2_ref[...], 0.0)
    sl = jnp.dot(h, wp_ref[...], preferred_element_type=jnp.float32) \
        + bp_ref[...]
    m = jnp.max(sl, axis=1, keepdims=True)
    p = jnp.exp(sl - m)
    s = p / jnp.sum(p, axis=1, keepdims=True)
    st_ref[...] = jnp.concatenate(
        [s, jnp.ones((BL, 1), jnp.float32), jnp.zeros((BL, 5), jnp.float32)],
        axis=1)
    ssb = lax.dot_general(s, s, (((0,), (0,)), ((), ())),
                          preferred_element_type=jnp.float32)

    @pl.when(i == 0)
    def _():
        ss_ref[...] = jnp.zeros_like(ss_ref)

    ss_ref[...] += ssb
    lo = jnp.dot(h, wc_ref[...], preferred_element_type=jnp.float32) \
        + bc_ref[...]
    mm = jnp.max(lo, axis=1, keepdims=True)
    lp_ref[...] = lo - mm - jnp.log(
        jnp.sum(jnp.exp(lo - mm), axis=1, keepdims=True))


def _tc_d(ap, u2, dis, b2r, wp, bpr, wc, bcr):
    return pl.pallas_call(
        _tc_d_body,
        grid=(GRID,),
        in_specs=[
            pl.BlockSpec((NC, BL, H), lambda i: (0, i, 0)),
            pl.BlockSpec((BL, H), lambda i: (i, 0)),
            pl.BlockSpec((BL, L), lambda i: (i, 0)),
            pl.BlockSpec((1, H), lambda i: (0, 0)),
            pl.BlockSpec((H, C), lambda i: (0, 0)),
            pl.BlockSpec((1, C), lambda i: (0, 0)),
            pl.BlockSpec((H, NCLS), lambda i: (0, 0)),
            pl.BlockSpec((1, NCLS), lambda i: (0, 0)),
        ],
        out_specs=[
            pl.BlockSpec((BL, L), lambda i: (i, 0)),
            pl.BlockSpec((C, C), lambda i: (0, 0)),
            pl.BlockSpec((BL, NCLS), lambda i: (i, 0)),
        ],
        out_shape=[
            jax.ShapeDtypeStruct((N, L), jnp.float32),
            jax.ShapeDtypeStruct((C, C), jnp.float32),
            jax.ShapeDtypeStruct((N, NCLS), jnp.float32),
        ],
    )(ap, u2, dis, b2r, wp, bpr, wc, bcr)


def _tc_f_body(na_ref, da_ref, ss_ref, mc_ref, o_ref):
    num = jnp.sum(na_ref[...]) - float(E)
    den = jnp.sum(da_ref[...]) - float(E)
    mc_ref[...] = jnp.full((1, 1), -(num / den), jnp.float32)
    ssv = ss_ref[...]
    nss = jnp.sqrt(jnp.sum(ssv * ssv))
    eye = (lax.broadcasted_iota(jnp.int32, (C, C), 0)
           == lax.broadcasted_iota(jnp.int32, (C, C), 1)).astype(jnp.float32)
    t = ssv / nss - eye / jnp.sqrt(jnp.float32(C))
    o_ref[...] = jnp.full((1, 1), jnp.sqrt(jnp.sum(t * t)), jnp.float32)


def _tc_f(na, da, ss):
    return pl.pallas_call(
        _tc_f_body,
        out_shape=[
            jax.ShapeDtypeStruct((1, 1), jnp.float32),
            jax.ShapeDtypeStruct((1, 1), jnp.float32),
        ],
    )(na, da, ss)


def kernel(x, edge_index, W1, b1, W2, b2, Wp, bp, Wc, bc):
    src = edge_index[0]
    dst = edge_index[1]
    pad = jnp.zeros((EP - E,), jnp.int32)
    srcp = jnp.concatenate([src, pad]).reshape(NW, CH, CSZ)
    dstp = jnp.concatenate([dst, pad]).reshape(NW, CH, CSZ)

    degp = _sc_deg(dstp).reshape(NC, NPAD, L)
    u1, dis = _tc_b(x, W1, degp)

    a1 = _sc_agg(u1, srcp, dstp).reshape(NC, NPAD, H)
    u2 = _tc_c(a1, u1, dis, b1.reshape(1, H), W2)

    a2 = _sc_agg(u2, srcp, dstp).reshape(NC, NPAD, H)
    st, ss, logp = _tc_d(a2, u2, dis, b2.reshape(1, H),
                         Wp, bp.reshape(1, C), Wc, bc.reshape(1, NCLS))

    eacc = _sc_edge(st, srcp, dstp).reshape(NW, 2, L)
    mc, o = _tc_f(eacc[:, 0], eacc[:, 1], ss)
    return (logp, jnp.reshape(mc, ()), jnp.reshape(o, ()))


# speedup vs baseline: 26.7798x; 1.0823x over previous
import functools

import jax
import jax.numpy as jnp
from jax import lax
from jax.experimental import pallas as pl
from jax.experimental.pallas import tpu as pltpu
from jax.experimental.pallas import tpu_sc as plsc

N = 10000
DIN = 128
H = 32
C = 10
NCLS = 7
E = 160000

NC = 2
NS = 16
NW = NC * NS
L = 16

NPAD = 10112
RPW = NPAD // NS
CSZ = 125
CH = 40
EPT = CH * CSZ

BL = 1000
GRID = N // BL

_MESH = dict(core_axis_name="c", subcore_axis_name="s", num_cores=NC,
             num_subcores=NS)
_SC_PARAMS = pltpu.CompilerParams(use_tc_tiling_on_sc=False)


def _make_sc_deg():
    mesh = plsc.VectorSubcoreMesh(**_MESH)

    @functools.partial(
        pl.kernel,
        out_type=jax.ShapeDtypeStruct((NC * NPAD, L), jnp.float32),
        mesh=mesh,
        compiler_params=_SC_PARAMS,
        scratch_types=[
            pltpu.VMEM((CH, CSZ), jnp.int32),
            pltpu.VMEM((CSZ, L), jnp.float32),
            pltpu.VMEM((RPW, L), jnp.float32),
            pltpu.VMEM_SHARED((NPAD, L), jnp.float32),
            pltpu.SemaphoreType.DMA,
        ],
    )
    def k(ei_hbm, out_hbm, si_v, ones_v, stage_v, acc_sh, sem):
        cid = lax.axis_index("c")
        sid = lax.axis_index("s")
        wid = sid * NC + cid
        pltpu.sync_copy(ei_hbm.at[1, wid], si_v)
        one = jnp.full((L,), 1.0, jnp.float32)
        zero = jnp.zeros((L,), jnp.float32)

        def f1(i, _):
            ones_v[i, :] = one
            return 0

        lax.fori_loop(0, CSZ, f1, 0)

        def fz(i, _):
            stage_v[i, :] = zero
            return 0

        lax.fori_loop(0, RPW, fz, 0)
        pltpu.sync_copy(stage_v, acc_sh.at[pl.ds(sid * RPW, RPW)])
        plsc.subcore_barrier()

        def fire(j, _):
            pltpu.async_copy(ones_v, acc_sh.at[si_v.at[j]], sem, add=True)
            return 0

        lax.fori_loop(0, CH, fire, 0)

        def drain(j, _):
            pltpu.make_async_copy(ones_v, acc_sh.at[si_v.at[j]], sem).wait()
            return 0

        lax.fori_loop(0, CH, drain, 0)
        plsc.subcore_barrier()
        pltpu.sync_copy(
            acc_sh.at[pl.ds(sid * RPW, RPW)],
            out_hbm.at[pl.ds(cid * NPAD + sid * RPW, RPW)],
        )

    return k


def _make_sc_agg(W):
    mesh = plsc.VectorSubcoreMesh(**_MESH)
    NB = 4

    @functools.partial(
        pl.kernel,
        out_type=jax.ShapeDtypeStruct((NC * NPAD, W), jnp.float32),
        mesh=mesh,
        compiler_params=_SC_PARAMS,
        scratch_types=[
            pltpu.VMEM((CH, CSZ), jnp.int32),
            pltpu.VMEM((CH, CSZ), jnp.int32),
        ] + [pltpu.VMEM((CSZ, W), jnp.float32) for _ in range(NB)] + [
            pltpu.VMEM((RPW, W), jnp.float32),
            pltpu.VMEM_SHARED((NPAD, W), jnp.float32),
        ] + [pltpu.SemaphoreType.DMA for _ in range(2 * NB)],
    )
    def k(table_hbm, ei_hbm, out_hbm, gi_v, si_v, r0, r1, r2, r3,
          stage_v, acc_sh, sg0, sg1, sg2, sg3, ss0, ss1, ss2, ss3):
        cid = lax.axis_index("c")
        sid = lax.axis_index("s")
        wid = sid * NC + cid
        bufs = (r0, r1, r2, r3)
        gsems = (sg0, sg1, sg2, sg3)
        ssems = (ss0, ss1, ss2, ss3)
        pltpu.sync_copy(ei_hbm.at[0, wid], gi_v)
        pltpu.sync_copy(ei_hbm.at[1, wid], si_v)
        zero = jnp.zeros((L,), jnp.float32)

        def fz(i, _):
            for w0 in range(W // L):
                stage_v[i, pl.ds(w0 * L, L)] = zero
            return 0

        lax.fori_loop(0, RPW, fz, 0)
        pltpu.sync_copy(stage_v, acc_sh.at[pl.ds(sid * RPW, RPW)])
        plsc.subcore_barrier()

        def body(t, _):
            j = NB * t
            gathers = [
                pltpu.async_copy(table_hbm.at[gi_v.at[j + b]], bufs[b],
                                 gsems[b])
                for b in range(NB)
            ]
            scatters = []
            for b in range(NB):
                gathers[b].wait()
                scatters.append(
                    pltpu.async_copy(bufs[b], acc_sh.at[si_v.at[j + b]],
                                     ssems[b], add=True))
            for b in range(NB):
                scatters[b].wait()
            return 0

        lax.fori_loop(0, CH // NB, body, 0)
        plsc.subcore_barrier()
        pltpu.sync_copy(
            acc_sh.at[pl.ds(sid * RPW, RPW)],
            out_hbm.at[pl.ds(cid * NPAD + sid * RPW, RPW)],
        )

    return k


def _make_sc_edge():
    mesh = plsc.VectorSubcoreMesh(**_MESH)

    @functools.partial(
        pl.kernel,
        out_type=jax.ShapeDtypeStruct((NW * 2, L), jnp.float32),
        mesh=mesh,
        compiler_params=_SC_PARAMS,
        scratch_types=[
            pltpu.VMEM((CH, CSZ), jnp.int32),
            pltpu.VMEM((CH, CSZ), jnp.int32),
            pltpu.VMEM((CSZ, L), jnp.float32),
            pltpu.VMEM((CSZ, L), jnp.float32),
            pltpu.VMEM((CSZ, L), jnp.float32),
            pltpu.VMEM((CSZ, L), jnp.float32),
            pltpu.VMEM((2, L), jnp.float32),
            pltpu.SemaphoreType.DMA,
            pltpu.SemaphoreType.DMA,
        ],
    )
    def k(st_hbm, ei_hbm, out_hbm, si_v, di_v, ra0, rb0, ra1, rb1,
          res_v, sem0, sem1):
        cid = lax.axis_index("c")
        sid = lax.axis_index("s")
        wid = sid * NC + cid
        pltpu.sync_copy(ei_hbm.at[0, wid], si_v)
        pltpu.sync_copy(ei_hbm.at[1, wid], di_v)
        zero = jnp.zeros((L,), jnp.float32)

        def accum(ra, rb, carry):
            def inner(r, c2):
                accs = list(c2)
                base = r * 5
                for u in range(5):
                    va = ra[base + u]
                    vb = rb[base + u]
                    p = u % 4
                    accs[p] = accs[p] + va * vb
                    accs[4 + p] = accs[4 + p] + va * va
                return tuple(accs)

            return lax.fori_loop(0, CSZ // 5, inner, carry)

        def body(t, carry):
            j0 = 2 * t
            j1 = j0 + 1
            cpa0 = pltpu.async_copy(st_hbm.at[si_v.at[j0]], ra0, sem0)
            cpb0 = pltpu.async_copy(st_hbm.at[di_v.at[j0]], rb0, sem0)
            cpa1 = pltpu.async_copy(st_hbm.at[si_v.at[j1]], ra1, sem1)
            cpb1 = pltpu.async_copy(st_hbm.at[di_v.at[j1]], rb1, sem1)
            cpa0.wait()
            cpb0.wait()
            carry = accum(ra0, rb0, carry)
            cpa1.wait()
            cpb1.wait()
            return accum(ra1, rb1, carry)

        init = tuple(zero for _ in range(8))
        accs = lax.fori_loop(0, CH // 2, body, init)
        res_v[0] = accs[0] + accs[1] + accs[2] + accs[3]
        res_v[1] = accs[4] + accs[5] + accs[6] + accs[7]
        pltpu.sync_copy(res_v, out_hbm.at[pl.ds(wid * 2, 2)])

    return k


_sc_deg = _make_sc_deg()
_sc_agg = _make_sc_agg(H)
_sc_edge = _make_sc_edge()


def _tc_b_body(x_ref, w1_ref, dp_ref, u1_ref, dis_ref):
    deg = dp_ref[0] + dp_ref[1] + 1.0
    dis = lax.rsqrt(deg)
    h1 = jnp.dot(x_ref[...], w1_ref[...], preferred_element_type=jnp.float32)
    u1_ref[...] = dis[:, 0:1] * h1
    dis_ref[...] = dis


def _tc_b(x, w1, degp):
    return pl.pallas_call(
        _tc_b_body,
        grid=(GRID,),
        in_specs=[
            pl.BlockSpec((BL, DIN), lambda i: (i, 0)),
            pl.BlockSpec((DIN, H), lambda i: (0, 0)),
            pl.BlockSpec((NC, BL, L), lambda i: (0, i, 0)),
        ],
        out_specs=[
            pl.BlockSpec((BL, H), lambda i: (i, 0)),
            pl.BlockSpec((BL, L), lambda i: (i, 0)),
        ],
        out_shape=[
            jax.ShapeDtypeStruct((N, H), jnp.float32),
            jax.ShapeDtypeStruct((N, L), jnp.float32),
        ],
    )(x, w1, degp)


def _tc_c_body(ap_ref, u1_ref, dis_ref, b1_ref, w2_ref, u2_ref):
    d0 = dis_ref[...][:, 0:1]
    h = jnp.maximum(
        d0 * (ap_ref[0] + ap_ref[1] + u1_ref[...]) + b1_ref[...], 0.0)
    u2_ref[...] = d0 * jnp.dot(h, w2_ref[...],
                               preferred_element_type=jnp.float32)


def _tc_c(ap, u1, dis, b1r, w2):
    return pl.pallas_call(
        _tc_c_body,
        grid=(GRID,),
        in_specs=[
            pl.BlockSpec((NC, BL, H), lambda i: (0, i, 0)),
            pl.BlockSpec((BL, H), lambda i: (i, 0)),
            pl.BlockSpec((BL, L), lambda i: (i, 0)),
            pl.BlockSpec((1, H), lambda i: (0, 0)),
            pl.BlockSpec((H, H), lambda i: (0, 0)),
        ],
        out_specs=pl.BlockSpec((BL, H), lambda i: (i, 0)),
        out_shape=jax.ShapeDtypeStruct((N, H), jnp.float32),
    )(ap, u1, dis, b1r, w2)


def _tc_d_body(ap_ref, u2_ref, dis_ref, b2_ref, wp_ref, bp_ref,
               wc_ref, bc_ref, st_ref, ss_ref, lp_ref):
    i = pl.program_id(0)
    d0 = dis_ref[...][:, 0:1]
    h = jnp.maximum(
        d0 * (ap_ref[0] + ap_ref[1] + u2_ref[...]) + b2_ref[...], 0.0)
    sl = jnp.dot(h, wp_ref[...], preferred_element_type=jnp.float32) \
        + bp_ref[...]
    m = jnp.max(sl, axis=1, keepdims=True)
    p = jnp.exp(sl - m)
    s = p / jnp.sum(p, axis=1, keepdims=True)
    st_ref[...] = jnp.concatenate(
        [s, jnp.ones((BL, 1), jnp.float32), jnp.zeros((BL, 5), jnp.float32)],
        axis=1)
    ssb = lax.dot_general(s, s, (((0,), (0,)), ((), ())),
                          preferred_element_type=jnp.float32)

    @pl.when(i == 0)
    def _():
        ss_ref[...] = jnp.zeros_like(ss_ref)

    ss_ref[...] += ssb
    lo = jnp.dot(h, wc_ref[...], preferred_element_type=jnp.float32) \
        + bc_ref[...]
    mm = jnp.max(lo, axis=1, keepdims=True)
    lp_ref[...] = lo - mm - jnp.log(
        jnp.sum(jnp.exp(lo - mm), axis=1, keepdims=True))


def _tc_d(ap, u2, dis, b2r, wp, bpr, wc, bcr):
    return pl.pallas_call(
        _tc_d_body,
        grid=(GRID,),
        in_specs=[
            pl.BlockSpec((NC, BL, H), lambda i: (0, i, 0)),
            pl.BlockSpec((BL, H), lambda i: (i, 0)),
            pl.BlockSpec((BL, L), lambda i: (i, 0)),
            pl.BlockSpec((1, H), lambda i: (0, 0)),
            pl.BlockSpec((H, C), lambda i: (0, 0)),
            pl.BlockSpec((1, C), lambda i: (0, 0)),
            pl.BlockSpec((H, NCLS), lambda i: (0, 0)),
            pl.BlockSpec((1, NCLS), lambda i: (0, 0)),
        ],
        out_specs=[
            pl.BlockSpec((BL, L), lambda i: (i, 0)),
            pl.BlockSpec((C, C), lambda i: (0, 0)),
            pl.BlockSpec((BL, NCLS), lambda i: (i, 0)),
        ],
        out_shape=[
            jax.ShapeDtypeStruct((N, L), jnp.float32),
            jax.ShapeDtypeStruct((C, C), jnp.float32),
            jax.ShapeDtypeStruct((N, NCLS), jnp.float32),
        ],
    )(ap, u2, dis, b2r, wp, bpr, wc, bcr)


def _tc_f_body(na_ref, da_ref, ss_ref, mc_ref, o_ref):
    num = jnp.sum(na_ref[...]) - float(E)
    den = jnp.sum(da_ref[...]) - float(E)
    mc_ref[...] = jnp.full((1, 1), -(num / den), jnp.float32)
    ssv = ss_ref[...]
    nss = jnp.sqrt(jnp.sum(ssv * ssv))
    eye = (lax.broadcasted_iota(jnp.int32, (C, C), 0)
           == lax.broadcasted_iota(jnp.int32, (C, C), 1)).astype(jnp.float32)
    t = ssv / nss - eye / jnp.sqrt(jnp.float32(C))
    o_ref[...] = jnp.full((1, 1), jnp.sqrt(jnp.sum(t * t)), jnp.float32)


def _tc_f(na, da, ss):
    return pl.pallas_call(
        _tc_f_body,
        out_shape=[
            jax.ShapeDtypeStruct((1, 1), jnp.float32),
            jax.ShapeDtypeStruct((1, 1), jnp.float32),
        ],
    )(na, da, ss)


def kernel(x, edge_index, W1, b1, W2, b2, Wp, bp, Wc, bc):
    ei = edge_index.reshape(2, NW, CH, CSZ)

    degp = _sc_deg(ei).reshape(NC, NPAD, L)
    u1, dis = _tc_b(x, W1, degp)

    a1 = _sc_agg(u1, ei).reshape(NC, NPAD, H)
    u2 = _tc_c(a1, u1, dis, b1.reshape(1, H), W2)

    a2 = _sc_agg(u2, ei).reshape(NC, NPAD, H)
    st, ss, logp = _tc_d(a2, u2, dis, b2.reshape(1, H),
                         Wp, bp.reshape(1, C), Wc, bc.reshape(1, NCLS))

    eacc = _sc_edge(st, ei).reshape(NW, 2, L)
    mc, o = _tc_f(eacc[:, 0], eacc[:, 1], ss)
    return (logp, jnp.reshape(mc, ()), jnp.reshape(o, ()))
